# baseline (speedup 1.0000x reference)
import functools

import jax
import jax.numpy as jnp
from jax import lax
from jax.experimental import pallas as pl
from jax.experimental.pallas import tpu as pltpu
from jax.experimental.pallas import tpu_sc as plsc

B = 4
N = 8192
EMB = 768
ROWS = B * N
NC = 2
NS = 16
NW = NC * NS
RPW = ROWS // NW
K = 32
NCHUNK = RPW // K
NPAIR = NCHUNK // 2
LANES = 16
CPV = EMB // LANES

_mesh = plsc.VectorSubcoreMesh(core_axis_name="c", subcore_axis_name="s")


@functools.partial(
    pl.kernel,
    mesh=_mesh,
    out_type=jax.ShapeDtypeStruct((ROWS, EMB), jnp.float32),
    scratch_types=[
        pltpu.VMEM((RPW,), jnp.int32),
        pltpu.VMEM((K, EMB), jnp.float32),
        pltpu.VMEM((K, EMB), jnp.float32),
        pltpu.VMEM((K, EMB), jnp.float32),
        pltpu.VMEM((K, EMB), jnp.float32),
        pltpu.VMEM_SHARED((NS, K, EMB), jnp.float32),
        pltpu.SemaphoreType.DMA,
        pltpu.SemaphoreType.DMA,
        pltpu.SemaphoreType.DMA,
        pltpu.SemaphoreType.DMA,
        pltpu.SemaphoreType.DMA,
        pltpu.SemaphoreType.DMA,
        pltpu.SemaphoreType.DMA,
    ],
)
def _pos_emb_sc(x_hbm, idx_hbm, tab_hbm, out_hbm,
                idx_v, g0, g1, x0, x1, shared,
                gs0, gs1, xs0, xs1, cs, hs, o1):
    cid = lax.axis_index("c")
    sid = lax.axis_index("s")
    wid = sid * NC + cid
    base = wid * RPW
    pltpu.sync_copy(idx_hbm.at[pl.ds(base, RPW)], idx_v)

    def start_gather(g, gb, sem):
        pltpu.async_copy(tab_hbm.at[idx_v.at[pl.ds(g * K, K)]], gb, sem)

    def start_x(g, xb, sem):
        pltpu.async_copy(x_hbm.at[pl.ds(base + g * K, K)], xb, sem)

    def wait_loads(gb, xb, gsem, xsem):
        pltpu.make_async_copy(tab_hbm.at[idx_v.at[pl.ds(0, K)]], gb,
                              gsem).wait()
        pltpu.make_async_copy(x_hbm.at[pl.ds(base, K)], xb, xsem).wait()

    def wait_direct_out():
        pltpu.make_async_copy(x1, out_hbm.at[pl.ds(base, K)], o1).wait()

    def wait_spmem_copy():
        pltpu.make_async_copy(x0, shared.at[sid], cs).wait()

    def wait_hbm_write():
        pltpu.make_async_copy(shared.at[sid], out_hbm.at[pl.ds(base, K)],
                              hs).wait()

    def compute(gb, xb):
        def row_body(r, carry):
            for c in range(CPV):
                sl = pl.ds(c * LANES, LANES)
                plsc.addupdate(xb.at[r, sl], gb[r, sl])
            return carry
        lax.fori_loop(0, K, row_body, 0, unroll=2)

    start_gather(0, g0, gs0)
    start_x(0, x0, xs0)

    def pair_body(i, carry):
        a = 2 * i
        start_gather(a + 1, g1, gs1)

        @pl.when(i > 0)
        def _():
            wait_direct_out()
        start_x(a + 1, x1, xs1)
        wait_loads(g0, x0, gs0, xs0)
        compute(g0, x0)

        @pl.when(i > 0)
        def _():
            wait_hbm_write()
        pltpu.async_copy(x0, shared.at[sid], cs)

        @pl.when(i < NPAIR - 1)
        def _():
            start_gather(a + 2, g0, gs0)
        wait_spmem_copy()
        pltpu.async_copy(shared.at[sid],
                         out_hbm.at[pl.ds(base + a * K, K)], hs)

        @pl.when(i < NPAIR - 1)
        def _():
            start_x(a + 2, x0, xs0)
        wait_loads(g1, x1, gs1, xs1)
        compute(g1, x1)
        pltpu.async_copy(x1, out_hbm.at[pl.ds(base + (a + 1) * K, K)], o1)
        return carry

    lax.fori_loop(0, NPAIR, pair_body, 0)
    wait_hbm_write()
    wait_direct_out()


def kernel(x, apply_indices, pos_embedding):
    xf = x.reshape(ROWS, EMB)
    idx = apply_indices.reshape(ROWS).astype(jnp.int32)
    out = _pos_emb_sc(xf, idx, pos_embedding)
    return out.reshape(x.shape)


# speedup vs baseline: 1.0036x; 1.0036x over previous
import functools

import jax
import jax.numpy as jnp
from jax import lax
from jax.experimental import pallas as pl
from jax.experimental.pallas import tpu as pltpu
from jax.experimental.pallas import tpu_sc as plsc

B = 4
N = 8192
EMB = 768
ROWS = B * N
NC = 2
NS = 16
NW = NC * NS
RPW = ROWS // NW
K = 32
HK = K // 2
NCHUNK = RPW // K
NPAIR = NCHUNK // 2
LANES = 16
CPV = EMB // LANES

_mesh = plsc.VectorSubcoreMesh(core_axis_name="c", subcore_axis_name="s")


@functools.partial(
    pl.kernel,
    mesh=_mesh,
    out_type=jax.ShapeDtypeStruct((ROWS, EMB), jnp.float32),
    scratch_types=[
        pltpu.VMEM((RPW,), jnp.int32),
        pltpu.VMEM((K, EMB), jnp.float32),
        pltpu.VMEM((K, EMB), jnp.float32),
        pltpu.VMEM((K, EMB), jnp.float32),
        pltpu.VMEM((K, EMB), jnp.float32),
        pltpu.VMEM_SHARED((NS, 2, HK, EMB), jnp.float32),
        pltpu.SemaphoreType.DMA,
        pltpu.SemaphoreType.DMA,
        pltpu.SemaphoreType.DMA,
        pltpu.SemaphoreType.DMA,
        pltpu.SemaphoreType.DMA,
        pltpu.SemaphoreType.DMA,
        pltpu.SemaphoreType.DMA,
        pltpu.SemaphoreType.DMA,
    ],
)
def _pos_emb_sc(x_hbm, idx_hbm, tab_hbm, out_hbm,
                idx_v, g0, g1, x0, x1, shared,
                gs0, gs1, xs0, xs1, cs0, cs1, hs0, hs1):
    cid = lax.axis_index("c")
    sid = lax.axis_index("s")
    wid = sid * NC + cid
    base = wid * RPW
    pltpu.sync_copy(idx_hbm.at[pl.ds(base, RPW)], idx_v)

    cssems = (cs0, cs1)
    hssems = (hs0, hs1)

    def start_gather(g, gb, sem):
        pltpu.async_copy(tab_hbm.at[idx_v.at[pl.ds(g * K, K)]], gb, sem)

    def start_x(g, xb, sem):
        pltpu.async_copy(x_hbm.at[pl.ds(base + g * K, K)], xb, sem)

    def wait_loads(gb, xb, gsem, xsem):
        pltpu.make_async_copy(tab_hbm.at[idx_v.at[pl.ds(0, K)]], gb,
                              gsem).wait()
        pltpu.make_async_copy(x_hbm.at[pl.ds(base, K)], xb, xsem).wait()

    def wait_spmem_copy(h):
        pltpu.make_async_copy(x0.at[pl.ds(0, HK)], shared.at[sid, h],
                              cssems[h]).wait()

    def wait_hbm_write(h):
        pltpu.make_async_copy(shared.at[sid, h],
                              out_hbm.at[pl.ds(base, HK)], hssems[h]).wait()

    def write_chunk(g, xb, guard_first):
        for h in range(2):
            if guard_first:
                @pl.when(g > 0)
                def _():
                    wait_hbm_write(h)
            else:
                wait_hbm_write(h)
            pltpu.async_copy(xb.at[pl.ds(h * HK, HK)], shared.at[sid, h],
                             cssems[h])
        for h in range(2):
            wait_spmem_copy(h)
            pltpu.async_copy(shared.at[sid, h],
                             out_hbm.at[pl.ds(base + g * K + h * HK, HK)],
                             hssems[h])

    def compute(gb, xb):
        def row_body(r, carry):
            for c in range(CPV):
                sl = pl.ds(c * LANES, LANES)
                plsc.addupdate(xb.at[r, sl], gb[r, sl])
            return carry
        lax.fori_loop(0, K, row_body, 0, unroll=2)

    start_gather(0, g0, gs0)
    start_x(0, x0, xs0)

    def pair_body(i, carry):
        a = 2 * i
        start_gather(a + 1, g1, gs1)
        start_x(a + 1, x1, xs1)
        wait_loads(g0, x0, gs0, xs0)
        compute(g0, x0)
        write_chunk(a, x0, guard_first=True)

        @pl.when(i < NPAIR - 1)
        def _():
            start_gather(a + 2, g0, gs0)
            start_x(a + 2, x0, xs0)
        wait_loads(g1, x1, gs1, xs1)
        compute(g1, x1)
        write_chunk(a + 1, x1, guard_first=False)
        return carry

    lax.fori_loop(0, NPAIR, pair_body, 0)
    wait_hbm_write(0)
    wait_hbm_write(1)


def kernel(x, apply_indices, pos_embedding):
    xf = x.reshape(ROWS, EMB)
    idx = apply_indices.reshape(ROWS).astype(jnp.int32)
    out = _pos_emb_sc(xf, idx, pos_embedding)
    return out.reshape(x.shape)
